# baseline (speedup 1.0000x reference)
import functools

import jax
import jax.numpy as jnp
from jax import lax
from jax.experimental import pallas as pl
from jax.experimental.pallas import tpu as pltpu
from jax.experimental.pallas import tpu_sc as plsc

_NS_IN = 2562
_NS_OUT = 10242
_NALPHA = 6
_B = 8
_C = 128
_NBLK = 641
_NPAD = _NBLK * 16
_XW = _NALPHA * _NS_IN
_XWF = _XW + 4
_OW = _NALPHA * _NS_OUT
_OWB = _OW + 4

_NC = 2
_NS = 16
_NW = _NC * _NS
_U = _B * _C // 2
_UPW = _U // _NW

_P0A = 40960
_P0B = 61448 - _P0A


def _make_unpool():
    mesh = plsc.VectorSubcoreMesh(core_axis_name="c", subcore_axis_name="s")

    @functools.partial(
        pl.kernel,
        mesh=mesh,
        compiler_params=pltpu.CompilerParams(
            use_tc_tiling_on_sc=False, needs_layout_passes=False
        ),
        out_type=jax.ShapeDtypeStruct((_U, 2 * _OW), jnp.float32),
        scratch_types=[
            pltpu.VMEM((_NPAD,), jnp.int32),
            pltpu.VMEM((_NPAD,), jnp.int32),
            pltpu.VMEM((_XWF,), jnp.float32),
            pltpu.VMEM((_XWF,), jnp.float32),
            pltpu.VMEM((_OWB,), jnp.float32),
            pltpu.VMEM((8,), jnp.float32),
            pltpu.SemaphoreType.DMA,
            pltpu.SemaphoreType.DMA,
            pltpu.SemaphoreType.DMA,
            pltpu.SemaphoreType.DMA,
            pltpu.SemaphoreType.DMA,
            pltpu.SemaphoreType.DMA,
            pltpu.SemaphoreType.DMA,
        ],
    )
    def unpool(x_hbm, i0_hbm, i1_hbm, out_hbm,
               i0_v, i1_v, in0_v, in1_v, out_v, t8_v,
               si0, si1, s0a, s0b, st, s1a, s1b):
        wid = lax.axis_index("s") * _NC + lax.axis_index("c")
        u0 = wid * _UPW

        pltpu.sync_copy(i0_hbm, i0_v)
        pltpu.sync_copy(i1_hbm, i1_v)

        def in0(u):
            return pltpu.make_async_copy(
                x_hbm.at[u, pl.ds(0, _XWF)], in0_v, si0
            )

        def in1(u):
            return pltpu.make_async_copy(
                x_hbm.at[u, pl.ds(_XW - 4, _XWF)], in1_v, si1
            )

        def p0a(u):
            return pltpu.make_async_copy(
                out_v.at[pl.ds(0, _P0A)], out_hbm.at[u, pl.ds(0, _P0A)], s0a
            )

        def p0b(u):
            return pltpu.make_async_copy(
                out_v.at[pl.ds(_P0A, _P0B)],
                out_hbm.at[u, pl.ds(_P0A, _P0B)],
                s0b,
            )

        def tiny(u):
            return pltpu.make_async_copy(
                t8_v, out_hbm.at[u, pl.ds(_OW - 4, 8)], st
            )

        def p1a(u):
            return pltpu.make_async_copy(
                out_v.at[pl.ds(8, _P0A)],
                out_hbm.at[u, pl.ds(_OW + 4, _P0A)],
                s1a,
            )

        def p1b(u):
            return pltpu.make_async_copy(
                out_v.at[pl.ds(8 + _P0A, _P0B)],
                out_hbm.at[u, pl.ds(_OW + 4 + _P0A, _P0B)],
                s1b,
            )

        def gather_pass(in_ref, shift, alphas, head_store):
            @plsc.parallel_loop(0, _NBLK, unroll=4)
            def jblk(j):
                o = pl.multiple_of(j * 16, 16)
                i0 = i0_v[pl.ds(o, 16)]
                i1 = i1_v[pl.ds(o, 16)]
                ovec = lax.broadcasted_iota(jnp.int32, (16,), 0) + o
                m = ovec < _NS_OUT
                for a in alphas:
                    g0 = plsc.load_gather(in_ref, [i0 + (a * _NS_IN + shift)])
                    g1 = plsc.load_gather(in_ref, [i1 + (a * _NS_IN + shift)])
                    v = (g0 + g1) * 0.5
                    plsc.store_scatter(
                        out_v, [ovec + (a * _NS_OUT + shift)], v, mask=m
                    )
                    if head_store and a == 0:
                        plsc.store_scatter(
                            t8_v,
                            [jnp.minimum(ovec + 4, 7)],
                            v,
                            mask=ovec < 4,
                        )

        def do_pair(u, first, last):
            in0(u).wait()
            if not first:
                p1a(u).wait()
                p1b(u).wait()
            gather_pass(in0_v, 0, (0, 1, 2, 3), False)
            p0a(u).start()
            gather_pass(in0_v, 0, (4, 5), False)
            p0b(u).start()
            if not last:
                in0(u + 1).start()
            in1(u).wait()
            if not first:
                tiny(u).wait()
            lanes = lax.broadcasted_iota(jnp.int32, (16,), 0)
            tail = plsc.load_gather(
                out_v, [jnp.minimum(lanes + (_OW - 4), _OWB - 1)]
            )
            plsc.store_scatter(t8_v, [jnp.minimum(lanes, 7)], tail,
                               mask=lanes < 4)
            p0a(u).wait()
            p0b(u).wait()
            gather_pass(in1_v, 4, (0, 1, 2, 3), True)
            tiny(u).start()
            p1a(u).start()
            gather_pass(in1_v, 4, (4, 5), False)
            p1b(u).start()
            if not last:
                in1(u + 1).start()

        in0(u0).start()
        in1(u0).start()

        do_pair(u0, first=True, last=False)

        def outer(h, carry):
            do_pair(u0 + 1 + h, first=False, last=False)
            return carry

        lax.fori_loop(0, _UPW - 2, outer, 0)

        do_pair(u0 + _UPW - 1, first=False, last=True)
        tiny(u0 + _UPW - 1).wait()
        p1a(u0 + _UPW - 1).wait()
        p1b(u0 + _UPW - 1).wait()

    return unpool


_unpool = _make_unpool()


def kernel(x, index):
    idx = index.astype(jnp.int32)
    i0 = jnp.pad(idx[:_NS_OUT, 0] % _NS_IN, (0, _NPAD - _NS_OUT))
    i1 = jnp.pad(idx[:_NS_OUT, 1] % _NS_IN, (0, _NPAD - _NS_OUT))
    out = _unpool(x.reshape(_U, 2 * _XW), i0, i1)
    return out.reshape(_B, _C, _OW)


# speedup vs baseline: 2.8752x; 2.8752x over previous
import functools

import jax
import jax.numpy as jnp
from jax import lax
from jax.experimental import pallas as pl
from jax.experimental.pallas import tpu as pltpu
from jax.experimental.pallas import tpu_sc as plsc

_NS_IN = 2562
_NS_OUT = 10242
_NALPHA = 6
_B = 8
_C = 128
_NBLK = 641
_NPAD = _NBLK * 16
_XW = _NALPHA * _NS_IN
_XWF = _XW + 4
_OW = _NALPHA * _NS_OUT
_SPLIT = 4 * _NS_OUT

_NC = 2
_NS = 16
_NW = _NC * _NS
_P = _B * _C
_PPW = _P // _NW


def _make_unpool():
    mesh = plsc.VectorSubcoreMesh(core_axis_name="c", subcore_axis_name="s")

    @functools.partial(
        pl.kernel,
        mesh=mesh,
        compiler_params=pltpu.CompilerParams(
            use_tc_tiling_on_sc=False, needs_layout_passes=False
        ),
        out_type=jax.ShapeDtypeStruct((_B, _C, _OW), jnp.float32),
        scratch_types=[
            pltpu.VMEM((_NPAD,), jnp.int32),
            pltpu.VMEM((_NPAD,), jnp.int32),
            pltpu.VMEM((_XW,), jnp.float32),
            pltpu.VMEM((_XW,), jnp.float32),
            pltpu.VMEM((_OW,), jnp.float32),
            pltpu.SemaphoreType.DMA,
            pltpu.SemaphoreType.DMA,
            pltpu.SemaphoreType.DMA,
            pltpu.SemaphoreType.DMA,
        ],
    )
    def unpool(x_hbm, i0_hbm, i1_hbm, out_hbm,
               i0_v, i1_v, in0_v, in1_v, out_v, si0, si1, so1, so2):
        ins_v = (in0_v, in1_v)
        sems_i = (si0, si1)

        wid = lax.axis_index("s") * _NC + lax.axis_index("c")
        b = wid // 4
        c0 = (wid % 4) * _PPW

        pltpu.sync_copy(i0_hbm, i0_v)
        pltpu.sync_copy(i1_hbm, i1_v)

        def piece1(c):
            return pltpu.make_async_copy(
                out_v.at[pl.ds(0, _SPLIT)],
                out_hbm.at[b, c, pl.ds(0, _SPLIT)],
                so1,
            )

        def piece2(c):
            return pltpu.make_async_copy(
                out_v.at[pl.ds(_SPLIT, _OW - _SPLIT)],
                out_hbm.at[b, c, pl.ds(_SPLIT, _OW - _SPLIT)],
                so2,
            )

        def in_copy(s, c):
            return pltpu.make_async_copy(x_hbm.at[b, c], ins_v[s], sems_i[s])

        def gather_pass(s, alphas):
            @plsc.parallel_loop(0, _NBLK, unroll=4)
            def jblk(j):
                o = pl.multiple_of(j * 16, 16)
                i0 = i0_v[pl.ds(o, 16)]
                i1 = i1_v[pl.ds(o, 16)]
                ovec = lax.broadcasted_iota(jnp.int32, (16,), 0) + o
                m = ovec < _NS_OUT
                for a in alphas:
                    g0 = plsc.load_gather(ins_v[s], [i0 + a * _NS_IN])
                    g1 = plsc.load_gather(ins_v[s], [i1 + a * _NS_IN])
                    plsc.store_scatter(
                        out_v, [ovec + a * _NS_OUT], (g0 + g1) * 0.5, mask=m
                    )

        def iter_unit(s, c, first, prefetch):
            in_copy(s, c).wait()
            if not first:
                piece1(c).wait()
            gather_pass(s, (0, 1, 2, 3))
            piece1(c).start()
            if not first:
                piece2(c).wait()
            gather_pass(s, (4, 5))
            piece2(c).start()
            if prefetch:
                in_copy(s, c + 2).start()

        in_copy(0, c0).start()
        in_copy(1, c0 + 1).start()

        iter_unit(0, c0, first=True, prefetch=True)

        def outer(h, carry):
            k = 2 * h + 1
            iter_unit(1, c0 + k, first=False, prefetch=True)
            iter_unit(0, c0 + k + 1, first=False, prefetch=True)
            return carry

        lax.fori_loop(0, (_PPW - 4) // 2, outer, 0)

        iter_unit(1, c0 + _PPW - 3, first=False, prefetch=True)
        iter_unit(0, c0 + _PPW - 2, first=False, prefetch=False)
        iter_unit(1, c0 + _PPW - 1, first=False, prefetch=False)
        piece1(c0 + _PPW - 1).wait()
        piece2(c0 + _PPW - 1).wait()

    return unpool


_unpool = _make_unpool()


def kernel(x, index):
    idx = index.astype(jnp.int32)
    i0 = jnp.pad(idx[:_NS_OUT, 0] % _NS_IN, (0, _NPAD - _NS_OUT))
    i1 = jnp.pad(idx[:_NS_OUT, 1] % _NS_IN, (0, _NPAD - _NS_OUT))
    return _unpool(x, i0, i1)
